# baseline (speedup 1.0000x reference)
import functools

import jax
import jax.numpy as jnp
from jax import lax
from jax.experimental import pallas as pl
from jax.experimental.pallas import tpu as pltpu
from jax.experimental.pallas import tpu_sc as plsc

BATCH = 16384
EMB_DIM = 16
IN_FEAT = 6
HIDDEN = 64
PACK = 8
ROWW = 128
CHUNK = 128


def _sc_gather(cl128, seg128, cur128, crow, srow, urow):
    info = plsc.get_sparse_core_info()
    NC, NS = info.num_cores, info.num_subcores
    NW = NC * NS
    bpw = BATCH // NW
    nchunk = bpw // CHUNK

    mesh = plsc.VectorSubcoreMesh(core_axis_name="c", subcore_axis_name="s")

    @functools.partial(
        pl.kernel,
        mesh=mesh,
        out_type=[jax.ShapeDtypeStruct((BATCH, ROWW), jnp.float32)] * 3,
        scratch_types=[
            pltpu.VMEM((bpw,), jnp.int32),
            pltpu.VMEM((bpw,), jnp.int32),
            pltpu.VMEM((bpw,), jnp.int32),
            pltpu.VMEM((CHUNK, ROWW), jnp.float32),
            pltpu.VMEM((CHUNK, ROWW), jnp.float32),
            pltpu.SemaphoreType.DMA,
            pltpu.SemaphoreType.DMA,
        ],
    )
    def k(ce, se, ue, ci, si, ui, oc, osg, ocu, iv0, iv1, iv2, r0, r1, s0, s1):
        wid = lax.axis_index("s") * NC + lax.axis_index("c")
        base = wid * bpw
        pltpu.sync_copy(ci.at[pl.ds(base, bpw)], iv0)
        pltpu.sync_copy(si.at[pl.ds(base, bpw)], iv1)
        pltpu.sync_copy(ui.at[pl.ds(base, bpw)], iv2)
        tables = (ce, se, ue)
        ivs = (iv0, iv1, iv2)
        outs = (oc, osg, ocu)
        bufs = (r0, r1)
        sems = (s0, s1)
        jobs = [(t, p) for t in range(3) for p in range(nchunk)]

        def start(j, k_):
            t, p = jobs[k_]
            return pltpu.async_copy(
                tables[t].at[ivs[t].at[pl.ds(p * CHUNK, CHUNK)]],
                bufs[j % 2], sems[j % 2])

        pending = start(0, 0)
        for j in range(len(jobs)):
            nxt = start(j + 1, j + 1) if j + 1 < len(jobs) else None
            pending.wait()
            t, p = jobs[j]
            pltpu.sync_copy(bufs[j % 2],
                            outs[t].at[pl.ds(base + p * CHUNK, CHUNK)])
            pending = nxt

    return k(cl128, seg128, cur128, crow, srow, urow)


def _extract(rows, ids):
    off = lax.rem(ids, PACK)
    acc = jnp.where(off == 0, rows[:, 0:EMB_DIM], 0.0)
    for o in range(1, PACK):
        acc += jnp.where(off == o, rows[:, o * EMB_DIM:(o + 1) * EMB_DIM], 0.0)
    return acc


def _mlp_body(f_ref, ci_ref, si_ref, ui_ref, rc_ref, rs_ref, ru_ref,
              w1_ref, b1_ref, w2_ref, b2_ref, o_ref):
    xc = _extract(rc_ref[...], ci_ref[...])
    xs = _extract(rs_ref[...], si_ref[...])
    xu = _extract(ru_ref[...], ui_ref[...])
    h = jnp.dot(f_ref[...], w1_ref[0:IN_FEAT, :],
                preferred_element_type=jnp.float32)
    h += jnp.dot(xc, w1_ref[IN_FEAT:IN_FEAT + EMB_DIM, :],
                 preferred_element_type=jnp.float32)
    h += jnp.dot(xs, w1_ref[IN_FEAT + EMB_DIM:IN_FEAT + 2 * EMB_DIM, :],
                 preferred_element_type=jnp.float32)
    h += jnp.dot(xu, w1_ref[IN_FEAT + 2 * EMB_DIM:, :],
                 preferred_element_type=jnp.float32)
    h = jnp.maximum(h + b1_ref[...], 0.0)
    o_ref[...] = jnp.dot(h, w2_ref[...],
                         preferred_element_type=jnp.float32) + b2_ref[...]


def _mlp(features, cid, sid, uid, rc, rs, ru, W1, b1, W2, b2):
    BLK = 2048
    grid = (BATCH // BLK,)
    d_in = IN_FEAT + 3 * EMB_DIM
    out = pl.pallas_call(
        _mlp_body,
        grid=grid,
        in_specs=[
            pl.BlockSpec((BLK, IN_FEAT), lambda i: (i, 0)),
            pl.BlockSpec((BLK, 1), lambda i: (i, 0)),
            pl.BlockSpec((BLK, 1), lambda i: (i, 0)),
            pl.BlockSpec((BLK, 1), lambda i: (i, 0)),
            pl.BlockSpec((BLK, ROWW), lambda i: (i, 0)),
            pl.BlockSpec((BLK, ROWW), lambda i: (i, 0)),
            pl.BlockSpec((BLK, ROWW), lambda i: (i, 0)),
            pl.BlockSpec((d_in, HIDDEN), lambda i: (0, 0)),
            pl.BlockSpec((1, HIDDEN), lambda i: (0, 0)),
            pl.BlockSpec((HIDDEN, 1), lambda i: (0, 0)),
            pl.BlockSpec((1, 1), lambda i: (0, 0)),
        ],
        out_specs=pl.BlockSpec((BLK, 1), lambda i: (i, 0)),
        out_shape=jax.ShapeDtypeStruct((BATCH, 1), jnp.float32),
    )(features, cid.reshape(BATCH, 1), sid.reshape(BATCH, 1),
      uid.reshape(BATCH, 1), rc, rs, ru, W1, b1.reshape(1, HIDDEN), W2,
      b2.reshape(1, 1))
    return out[:, 0]


def kernel(features, client_id, segment_id, currency_pair_id,
           client_emb, segment_emb, currency_emb, W1, b1, W2, b2):
    cid = client_id.astype(jnp.int32)
    sid = segment_id.astype(jnp.int32)
    uid = currency_pair_id.astype(jnp.int32)
    cl128 = client_emb.reshape(-1, ROWW)
    seg128 = jnp.pad(segment_emb, ((0, 4), (0, 0))).reshape(-1, ROWW)
    cur128 = currency_emb.reshape(-1, ROWW)
    rc, rs, ru = _sc_gather(cl128, seg128, cur128,
                            cid // PACK, sid // PACK, uid // PACK)
    return _mlp(features, cid, sid, uid, rc, rs, ru, W1, b1, W2, b2)


# speedup vs baseline: 5.0428x; 5.0428x over previous
import functools

import jax
import jax.numpy as jnp
from jax import lax
from jax.experimental import pallas as pl
from jax.experimental.pallas import tpu as pltpu
from jax.experimental.pallas import tpu_sc as plsc

BATCH = 16384
EMB_DIM = 16
IN_FEAT = 6
HIDDEN = 64
NUM_CL = 1000000


def _tc_split(t_T):
    LBLK = 16384
    grid = (pl.cdiv(NUM_CL, LBLK),)

    def body(x_ref, *o_refs):
        x = x_ref[...]
        for d in range(EMB_DIM):
            o_refs[d][...] = x[d, :]

    return pl.pallas_call(
        body,
        grid=grid,
        in_specs=[pl.BlockSpec((EMB_DIM, LBLK), lambda i: (0, i))],
        out_specs=[pl.BlockSpec((LBLK,), lambda i: (i,))] * EMB_DIM,
        out_shape=[jax.ShapeDtypeStruct((NUM_CL,), jnp.float32)] * EMB_DIM,
    )(t_T)


def _sc_small(segment_emb, currency_emb, sid, uid):
    info = plsc.get_sparse_core_info()
    NC, NS = info.num_cores, info.num_subcores
    NW = NC * NS
    bpw = BATCH // NW

    mesh = plsc.VectorSubcoreMesh(core_axis_name="c", subcore_axis_name="s")

    @functools.partial(
        pl.kernel,
        mesh=mesh,
        out_type=[jax.ShapeDtypeStruct((BATCH, EMB_DIM), jnp.float32)] * 2,
        scratch_types=[
            pltpu.VMEM((bpw,), jnp.int32),
            pltpu.VMEM((bpw,), jnp.int32),
            pltpu.VMEM((bpw, EMB_DIM), jnp.float32),
            pltpu.VMEM((bpw, EMB_DIM), jnp.float32),
            pltpu.SemaphoreType.DMA,
            pltpu.SemaphoreType.DMA,
        ],
        compiler_params=pltpu.CompilerParams(use_tc_tiling_on_sc=False),
    )
    def k(se, ue, si, ui, osg, ocu, iv1, iv2, rv1, rv2, s1, s2):
        wid = lax.axis_index("s") * NC + lax.axis_index("c")
        base = wid * bpw
        pltpu.sync_copy(si.at[pl.ds(base, bpw)], iv1)
        pltpu.sync_copy(ui.at[pl.ds(base, bpw)], iv2)
        c1 = pltpu.async_copy(se.at[iv1], rv1, s1)
        c2 = pltpu.async_copy(ue.at[iv2], rv2, s2)
        c1.wait()
        c2.wait()
        pltpu.sync_copy(rv1, osg.at[pl.ds(base, bpw)])
        pltpu.sync_copy(rv2, ocu.at[pl.ds(base, bpw)])

    return k(segment_emb, currency_emb, sid, uid)


def _sc_client_elem(e_list, cid):
    info = plsc.get_sparse_core_info()
    NC, NS = info.num_cores, info.num_subcores
    NW = NC * NS
    bpw = BATCH // NW
    NSEM = 4

    mesh = plsc.VectorSubcoreMesh(core_axis_name="c", subcore_axis_name="s")

    @functools.partial(
        pl.kernel,
        mesh=mesh,
        out_type=[jax.ShapeDtypeStruct((BATCH,), jnp.float32)] * EMB_DIM,
        scratch_types=(
            [pltpu.VMEM((bpw,), jnp.int32)]
            + [pltpu.VMEM((bpw,), jnp.float32)] * EMB_DIM
            + [pltpu.SemaphoreType.DMA] * NSEM
        ),
        compiler_params=pltpu.CompilerParams(use_tc_tiling_on_sc=False),
    )
    def k(*refs):
        tables = refs[0:EMB_DIM]
        ci = refs[EMB_DIM]
        outs = refs[EMB_DIM + 1:2 * EMB_DIM + 1]
        iv = refs[2 * EMB_DIM + 1]
        dests = refs[2 * EMB_DIM + 2:3 * EMB_DIM + 2]
        sems = refs[3 * EMB_DIM + 2:]
        wid = lax.axis_index("s") * NC + lax.axis_index("c")
        base = wid * bpw
        pltpu.sync_copy(ci.at[pl.ds(base, bpw)], iv)
        copies = []
        for d in range(EMB_DIM):
            copies.append(
                pltpu.async_copy(tables[d].at[iv], dests[d], sems[d % NSEM]))
        for d in range(EMB_DIM):
            copies[d].wait()
            pltpu.sync_copy(dests[d], outs[d].at[pl.ds(base, bpw)])

    return k(*e_list, cid)


def _mlp_body(f_ref, s_ref, u_ref, *rest):
    rc_refs = rest[0:EMB_DIM]
    w1_ref, b1_ref, w2_ref, b2_ref, o_ref = rest[EMB_DIM:]
    xcT = jnp.concatenate([rc_refs[d][...][None, :] for d in range(EMB_DIM)],
                          axis=0)
    xc = jnp.transpose(xcT, (1, 0))
    h = jnp.dot(f_ref[...], w1_ref[0:IN_FEAT, :],
                preferred_element_type=jnp.float32)
    h += jnp.dot(xc, w1_ref[IN_FEAT:IN_FEAT + EMB_DIM, :],
                 preferred_element_type=jnp.float32)
    h += jnp.dot(s_ref[...], w1_ref[IN_FEAT + EMB_DIM:IN_FEAT + 2 * EMB_DIM, :],
                 preferred_element_type=jnp.float32)
    h += jnp.dot(u_ref[...], w1_ref[IN_FEAT + 2 * EMB_DIM:, :],
                 preferred_element_type=jnp.float32)
    h = jnp.maximum(h + b1_ref[...], 0.0)
    o_ref[...] = jnp.dot(h, w2_ref[...],
                         preferred_element_type=jnp.float32) + b2_ref[...]


def _mlp(features, rc_list, rs, ru, W1, b1, W2, b2):
    BLK = 2048
    grid = (BATCH // BLK,)
    d_in = IN_FEAT + 3 * EMB_DIM
    out = pl.pallas_call(
        _mlp_body,
        grid=grid,
        in_specs=(
            [
                pl.BlockSpec((BLK, IN_FEAT), lambda i: (i, 0)),
                pl.BlockSpec((BLK, EMB_DIM), lambda i: (i, 0)),
                pl.BlockSpec((BLK, EMB_DIM), lambda i: (i, 0)),
            ]
            + [pl.BlockSpec((BLK,), lambda i: (i,))] * EMB_DIM
            + [
                pl.BlockSpec((d_in, HIDDEN), lambda i: (0, 0)),
                pl.BlockSpec((1, HIDDEN), lambda i: (0, 0)),
                pl.BlockSpec((HIDDEN, 1), lambda i: (0, 0)),
                pl.BlockSpec((1, 1), lambda i: (0, 0)),
            ]
        ),
        out_specs=pl.BlockSpec((BLK, 1), lambda i: (i, 0)),
        out_shape=jax.ShapeDtypeStruct((BATCH, 1), jnp.float32),
    )(features, rs, ru, *rc_list, W1, b1.reshape(1, HIDDEN), W2,
      b2.reshape(1, 1))
    return out[:, 0]


def kernel(features, client_id, segment_id, currency_pair_id,
           client_emb, segment_emb, currency_emb, W1, b1, W2, b2):
    cid = client_id.astype(jnp.int32)
    sid = segment_id.astype(jnp.int32)
    uid = currency_pair_id.astype(jnp.int32)
    rs, ru = _sc_small(segment_emb, currency_emb, sid, uid)
    e_list = _tc_split(client_emb.T)
    rc_list = _sc_client_elem(e_list, cid)
    return _mlp(features, rc_list, rs, ru, W1, b1, W2, b2)


# speedup vs baseline: 5.7602x; 1.1423x over previous
import functools

import jax
import jax.numpy as jnp
from jax import lax
from jax.experimental import pallas as pl
from jax.experimental.pallas import tpu as pltpu
from jax.experimental.pallas import tpu_sc as plsc

BATCH = 16384
EMB_DIM = 16
IN_FEAT = 6
HIDDEN = 64
NUM_CL = 1000000


def _tc_split(t_T):
    LBLK = 32768
    grid = (pl.cdiv(NUM_CL, LBLK),)

    def body(x_ref, *o_refs):
        x = x_ref[...]
        for d in range(EMB_DIM):
            o_refs[d][...] = x[d, :]

    return pl.pallas_call(
        body,
        grid=grid,
        in_specs=[pl.BlockSpec((EMB_DIM, LBLK), lambda i: (0, i))],
        out_specs=[pl.BlockSpec((LBLK,), lambda i: (i,))] * EMB_DIM,
        out_shape=[jax.ShapeDtypeStruct((NUM_CL,), jnp.float32)] * EMB_DIM,
    )(t_T)


def _sc_small(segment_emb, currency_emb, sid, uid):
    info = plsc.get_sparse_core_info()
    NC, NS = info.num_cores, info.num_subcores
    NW = NC * NS
    bpw = BATCH // NW

    mesh = plsc.VectorSubcoreMesh(core_axis_name="c", subcore_axis_name="s")

    @functools.partial(
        pl.kernel,
        mesh=mesh,
        out_type=[jax.ShapeDtypeStruct((BATCH, EMB_DIM), jnp.float32)] * 2,
        scratch_types=[
            pltpu.VMEM((bpw,), jnp.int32),
            pltpu.VMEM((bpw,), jnp.int32),
            pltpu.VMEM((bpw, EMB_DIM), jnp.float32),
            pltpu.VMEM((bpw, EMB_DIM), jnp.float32),
            pltpu.SemaphoreType.DMA,
            pltpu.SemaphoreType.DMA,
        ],
        compiler_params=pltpu.CompilerParams(use_tc_tiling_on_sc=False),
    )
    def k(se, ue, si, ui, osg, ocu, iv1, iv2, rv1, rv2, s1, s2):
        wid = lax.axis_index("s") * NC + lax.axis_index("c")
        base = wid * bpw
        pltpu.sync_copy(si.at[pl.ds(base, bpw)], iv1)
        pltpu.sync_copy(ui.at[pl.ds(base, bpw)], iv2)
        c1 = pltpu.async_copy(se.at[iv1], rv1, s1)
        c2 = pltpu.async_copy(ue.at[iv2], rv2, s2)
        c1.wait()
        c2.wait()
        pltpu.sync_copy(rv1, osg.at[pl.ds(base, bpw)])
        pltpu.sync_copy(rv2, ocu.at[pl.ds(base, bpw)])

    return k(segment_emb, currency_emb, sid, uid)


def _sc_client_elem(e_list, cid):
    info = plsc.get_sparse_core_info()
    NC, NS = info.num_cores, info.num_subcores
    NW = NC * NS
    bpw = BATCH // NW
    NSEM = 4

    mesh = plsc.VectorSubcoreMesh(core_axis_name="c", subcore_axis_name="s")

    @functools.partial(
        pl.kernel,
        mesh=mesh,
        out_type=[jax.ShapeDtypeStruct((BATCH,), jnp.float32)] * EMB_DIM,
        scratch_types=(
            [pltpu.VMEM((bpw,), jnp.int32)]
            + [pltpu.VMEM((bpw,), jnp.float32)] * EMB_DIM
            + [pltpu.SemaphoreType.DMA] * NSEM
        ),
        compiler_params=pltpu.CompilerParams(use_tc_tiling_on_sc=False),
    )
    def k(*refs):
        tables = refs[0:EMB_DIM]
        ci = refs[EMB_DIM]
        outs = refs[EMB_DIM + 1:2 * EMB_DIM + 1]
        iv = refs[2 * EMB_DIM + 1]
        dests = refs[2 * EMB_DIM + 2:3 * EMB_DIM + 2]
        sems = refs[3 * EMB_DIM + 2:]
        wid = lax.axis_index("s") * NC + lax.axis_index("c")
        base = wid * bpw
        pltpu.sync_copy(ci.at[pl.ds(base, bpw)], iv)
        copies = []
        for d in range(EMB_DIM):
            copies.append(
                pltpu.async_copy(tables[d].at[iv], dests[d], sems[d % NSEM]))
        for d in range(EMB_DIM):
            copies[d].wait()
            pltpu.sync_copy(dests[d], outs[d].at[pl.ds(base, bpw)])

    return k(*e_list, cid)


def _mlp_body(f_ref, s_ref, u_ref, *rest):
    rc_refs = rest[0:EMB_DIM]
    w1_ref, b1_ref, w2_ref, b2_ref, o_ref = rest[EMB_DIM:]
    xcT = jnp.concatenate([rc_refs[d][...][None, :] for d in range(EMB_DIM)],
                          axis=0)
    xc = jnp.transpose(xcT, (1, 0))
    h = jnp.dot(f_ref[...], w1_ref[0:IN_FEAT, :],
                preferred_element_type=jnp.float32)
    h += jnp.dot(xc, w1_ref[IN_FEAT:IN_FEAT + EMB_DIM, :],
                 preferred_element_type=jnp.float32)
    h += jnp.dot(s_ref[...], w1_ref[IN_FEAT + EMB_DIM:IN_FEAT + 2 * EMB_DIM, :],
                 preferred_element_type=jnp.float32)
    h += jnp.dot(u_ref[...], w1_ref[IN_FEAT + 2 * EMB_DIM:, :],
                 preferred_element_type=jnp.float32)
    h = jnp.maximum(h + b1_ref[...], 0.0)
    o_ref[...] = jnp.dot(h, w2_ref[...],
                         preferred_element_type=jnp.float32) + b2_ref[...]


def _mlp(features, rc_list, rs, ru, W1, b1, W2, b2):
    BLK = 2048
    grid = (BATCH // BLK,)
    d_in = IN_FEAT + 3 * EMB_DIM
    out = pl.pallas_call(
        _mlp_body,
        grid=grid,
        in_specs=(
            [
                pl.BlockSpec((BLK, IN_FEAT), lambda i: (i, 0)),
                pl.BlockSpec((BLK, EMB_DIM), lambda i: (i, 0)),
                pl.BlockSpec((BLK, EMB_DIM), lambda i: (i, 0)),
            ]
            + [pl.BlockSpec((BLK,), lambda i: (i,))] * EMB_DIM
            + [
                pl.BlockSpec((d_in, HIDDEN), lambda i: (0, 0)),
                pl.BlockSpec((1, HIDDEN), lambda i: (0, 0)),
                pl.BlockSpec((HIDDEN, 1), lambda i: (0, 0)),
                pl.BlockSpec((1, 1), lambda i: (0, 0)),
            ]
        ),
        out_specs=pl.BlockSpec((BLK, 1), lambda i: (i, 0)),
        out_shape=jax.ShapeDtypeStruct((BATCH, 1), jnp.float32),
    )(features, rs, ru, *rc_list, W1, b1.reshape(1, HIDDEN), W2,
      b2.reshape(1, 1))
    return out[:, 0]


def kernel(features, client_id, segment_id, currency_pair_id,
           client_emb, segment_emb, currency_emb, W1, b1, W2, b2):
    cid = client_id.astype(jnp.int32)
    sid = segment_id.astype(jnp.int32)
    uid = currency_pair_id.astype(jnp.int32)
    rs, ru = _sc_small(segment_emb, currency_emb, sid, uid)
    e_list = _tc_split(client_emb.T)
    rc_list = _sc_client_elem(e_list, cid)
    return _mlp(features, rc_list, rs, ru, W1, b1, W2, b2)


# speedup vs baseline: 6.0236x; 1.0457x over previous
import functools

import jax
import jax.numpy as jnp
from jax import lax
from jax.experimental import pallas as pl
from jax.experimental.pallas import tpu as pltpu
from jax.experimental.pallas import tpu_sc as plsc

BATCH = 16384
EMB_DIM = 16
IN_FEAT = 6
HIDDEN = 64
NUM_CL = 1000000


def _tc_split(t_T):
    LBLK = 65536
    grid = (pl.cdiv(NUM_CL, LBLK),)

    def body(x_ref, *o_refs):
        x = x_ref[...]
        for d in range(EMB_DIM):
            o_refs[d][...] = x[d, :]

    return pl.pallas_call(
        body,
        grid=grid,
        in_specs=[pl.BlockSpec((EMB_DIM, LBLK), lambda i: (0, i))],
        out_specs=[pl.BlockSpec((LBLK,), lambda i: (i,))] * EMB_DIM,
        out_shape=[jax.ShapeDtypeStruct((NUM_CL,), jnp.float32)] * EMB_DIM,
    )(t_T)


def _sc_small(segment_emb, currency_emb, sid, uid):
    info = plsc.get_sparse_core_info()
    NC, NS = info.num_cores, info.num_subcores
    NW = NC * NS
    bpw = BATCH // NW

    mesh = plsc.VectorSubcoreMesh(core_axis_name="c", subcore_axis_name="s")

    @functools.partial(
        pl.kernel,
        mesh=mesh,
        out_type=[jax.ShapeDtypeStruct((BATCH, EMB_DIM), jnp.float32)] * 2,
        scratch_types=[
            pltpu.VMEM((bpw,), jnp.int32),
            pltpu.VMEM((bpw,), jnp.int32),
            pltpu.VMEM((bpw, EMB_DIM), jnp.float32),
            pltpu.VMEM((bpw, EMB_DIM), jnp.float32),
            pltpu.SemaphoreType.DMA,
            pltpu.SemaphoreType.DMA,
        ],
        compiler_params=pltpu.CompilerParams(use_tc_tiling_on_sc=False),
    )
    def k(se, ue, si, ui, osg, ocu, iv1, iv2, rv1, rv2, s1, s2):
        wid = lax.axis_index("s") * NC + lax.axis_index("c")
        base = wid * bpw
        pltpu.sync_copy(si.at[pl.ds(base, bpw)], iv1)
        pltpu.sync_copy(ui.at[pl.ds(base, bpw)], iv2)
        c1 = pltpu.async_copy(se.at[iv1], rv1, s1)
        c2 = pltpu.async_copy(ue.at[iv2], rv2, s2)
        c1.wait()
        c2.wait()
        pltpu.sync_copy(rv1, osg.at[pl.ds(base, bpw)])
        pltpu.sync_copy(rv2, ocu.at[pl.ds(base, bpw)])

    return k(segment_emb, currency_emb, sid, uid)


def _sc_client_elem(e_list, cid):
    info = plsc.get_sparse_core_info()
    NC, NS = info.num_cores, info.num_subcores
    NW = NC * NS
    bpw = BATCH // NW
    NSEM = 4

    mesh = plsc.VectorSubcoreMesh(core_axis_name="c", subcore_axis_name="s")

    @functools.partial(
        pl.kernel,
        mesh=mesh,
        out_type=[jax.ShapeDtypeStruct((BATCH,), jnp.float32)] * EMB_DIM,
        scratch_types=(
            [pltpu.VMEM((bpw,), jnp.int32)]
            + [pltpu.VMEM((bpw,), jnp.float32)] * EMB_DIM
            + [pltpu.SemaphoreType.DMA] * NSEM
        ),
        compiler_params=pltpu.CompilerParams(use_tc_tiling_on_sc=False),
    )
    def k(*refs):
        tables = refs[0:EMB_DIM]
        ci = refs[EMB_DIM]
        outs = refs[EMB_DIM + 1:2 * EMB_DIM + 1]
        iv = refs[2 * EMB_DIM + 1]
        dests = refs[2 * EMB_DIM + 2:3 * EMB_DIM + 2]
        sems = refs[3 * EMB_DIM + 2:]
        wid = lax.axis_index("s") * NC + lax.axis_index("c")
        base = wid * bpw
        pltpu.sync_copy(ci.at[pl.ds(base, bpw)], iv)
        copies = []
        for d in range(EMB_DIM):
            copies.append(
                pltpu.async_copy(tables[d].at[iv], dests[d], sems[d % NSEM]))
        for d in range(EMB_DIM):
            copies[d].wait()
            pltpu.sync_copy(dests[d], outs[d].at[pl.ds(base, bpw)])

    return k(*e_list, cid)


def _mlp_body(f_ref, s_ref, u_ref, *rest):
    rc_refs = rest[0:EMB_DIM]
    w1_ref, b1_ref, w2_ref, b2_ref, o_ref = rest[EMB_DIM:]
    xcT = jnp.concatenate([rc_refs[d][...][None, :] for d in range(EMB_DIM)],
                          axis=0)
    xc = jnp.transpose(xcT, (1, 0))
    h = jnp.dot(f_ref[...], w1_ref[0:IN_FEAT, :],
                preferred_element_type=jnp.float32)
    h += jnp.dot(xc, w1_ref[IN_FEAT:IN_FEAT + EMB_DIM, :],
                 preferred_element_type=jnp.float32)
    h += jnp.dot(s_ref[...], w1_ref[IN_FEAT + EMB_DIM:IN_FEAT + 2 * EMB_DIM, :],
                 preferred_element_type=jnp.float32)
    h += jnp.dot(u_ref[...], w1_ref[IN_FEAT + 2 * EMB_DIM:, :],
                 preferred_element_type=jnp.float32)
    h = jnp.maximum(h + b1_ref[...], 0.0)
    o_ref[...] = jnp.dot(h, w2_ref[...],
                         preferred_element_type=jnp.float32) + b2_ref[...]


def _mlp(features, rc_list, rs, ru, W1, b1, W2, b2):
    BLK = 2048
    grid = (BATCH // BLK,)
    d_in = IN_FEAT + 3 * EMB_DIM
    out = pl.pallas_call(
        _mlp_body,
        grid=grid,
        in_specs=(
            [
                pl.BlockSpec((BLK, IN_FEAT), lambda i: (i, 0)),
                pl.BlockSpec((BLK, EMB_DIM), lambda i: (i, 0)),
                pl.BlockSpec((BLK, EMB_DIM), lambda i: (i, 0)),
            ]
            + [pl.BlockSpec((BLK,), lambda i: (i,))] * EMB_DIM
            + [
                pl.BlockSpec((d_in, HIDDEN), lambda i: (0, 0)),
                pl.BlockSpec((1, HIDDEN), lambda i: (0, 0)),
                pl.BlockSpec((HIDDEN, 1), lambda i: (0, 0)),
                pl.BlockSpec((1, 1), lambda i: (0, 0)),
            ]
        ),
        out_specs=pl.BlockSpec((BLK, 1), lambda i: (i, 0)),
        out_shape=jax.ShapeDtypeStruct((BATCH, 1), jnp.float32),
    )(features, rs, ru, *rc_list, W1, b1.reshape(1, HIDDEN), W2,
      b2.reshape(1, 1))
    return out[:, 0]


def kernel(features, client_id, segment_id, currency_pair_id,
           client_emb, segment_emb, currency_emb, W1, b1, W2, b2):
    cid = client_id.astype(jnp.int32)
    sid = segment_id.astype(jnp.int32)
    uid = currency_pair_id.astype(jnp.int32)
    rs, ru = _sc_small(segment_emb, currency_emb, sid, uid)
    e_list = _tc_split(client_emb.T)
    rc_list = _sc_client_elem(e_list, cid)
    return _mlp(features, rc_list, rs, ru, W1, b1, W2, b2)


# speedup vs baseline: 6.1843x; 1.0267x over previous
import functools

import jax
import jax.numpy as jnp
from jax import lax
from jax.experimental import pallas as pl
from jax.experimental.pallas import tpu as pltpu
from jax.experimental.pallas import tpu_sc as plsc

BATCH = 16384
EMB_DIM = 16
IN_FEAT = 6
HIDDEN = 64
NUM_CL = 1000000


def _tc_split(t_T):
    LBLK = 131072
    grid = (pl.cdiv(NUM_CL, LBLK),)

    def body(x_ref, *o_refs):
        x = x_ref[...]
        for d in range(EMB_DIM):
            o_refs[d][...] = x[d, :]

    return pl.pallas_call(
        body,
        grid=grid,
        in_specs=[pl.BlockSpec((EMB_DIM, LBLK), lambda i: (0, i))],
        out_specs=[pl.BlockSpec((LBLK,), lambda i: (i,))] * EMB_DIM,
        out_shape=[jax.ShapeDtypeStruct((NUM_CL,), jnp.float32)] * EMB_DIM,
    )(t_T)


def _sc_small(segment_emb, currency_emb, sid, uid):
    info = plsc.get_sparse_core_info()
    NC, NS = info.num_cores, info.num_subcores
    NW = NC * NS
    bpw = BATCH // NW

    mesh = plsc.VectorSubcoreMesh(core_axis_name="c", subcore_axis_name="s")

    @functools.partial(
        pl.kernel,
        mesh=mesh,
        out_type=[jax.ShapeDtypeStruct((BATCH, EMB_DIM), jnp.float32)] * 2,
        scratch_types=[
            pltpu.VMEM((bpw,), jnp.int32),
            pltpu.VMEM((bpw,), jnp.int32),
            pltpu.VMEM((bpw, EMB_DIM), jnp.float32),
            pltpu.VMEM((bpw, EMB_DIM), jnp.float32),
            pltpu.SemaphoreType.DMA,
            pltpu.SemaphoreType.DMA,
        ],
        compiler_params=pltpu.CompilerParams(use_tc_tiling_on_sc=False),
    )
    def k(se, ue, si, ui, osg, ocu, iv1, iv2, rv1, rv2, s1, s2):
        wid = lax.axis_index("s") * NC + lax.axis_index("c")
        base = wid * bpw
        pltpu.sync_copy(si.at[pl.ds(base, bpw)], iv1)
        pltpu.sync_copy(ui.at[pl.ds(base, bpw)], iv2)
        c1 = pltpu.async_copy(se.at[iv1], rv1, s1)
        c2 = pltpu.async_copy(ue.at[iv2], rv2, s2)
        c1.wait()
        c2.wait()
        pltpu.sync_copy(rv1, osg.at[pl.ds(base, bpw)])
        pltpu.sync_copy(rv2, ocu.at[pl.ds(base, bpw)])

    return k(segment_emb, currency_emb, sid, uid)


def _sc_client_elem(e_list, cid):
    info = plsc.get_sparse_core_info()
    NC, NS = info.num_cores, info.num_subcores
    NW = NC * NS
    bpw = BATCH // NW
    NSEM = 4

    mesh = plsc.VectorSubcoreMesh(core_axis_name="c", subcore_axis_name="s")

    @functools.partial(
        pl.kernel,
        mesh=mesh,
        out_type=[jax.ShapeDtypeStruct((BATCH,), jnp.float32)] * EMB_DIM,
        scratch_types=(
            [pltpu.VMEM((bpw,), jnp.int32)]
            + [pltpu.VMEM((bpw,), jnp.float32)] * EMB_DIM
            + [pltpu.SemaphoreType.DMA] * NSEM
        ),
        compiler_params=pltpu.CompilerParams(use_tc_tiling_on_sc=False),
    )
    def k(*refs):
        tables = refs[0:EMB_DIM]
        ci = refs[EMB_DIM]
        outs = refs[EMB_DIM + 1:2 * EMB_DIM + 1]
        iv = refs[2 * EMB_DIM + 1]
        dests = refs[2 * EMB_DIM + 2:3 * EMB_DIM + 2]
        sems = refs[3 * EMB_DIM + 2:]
        wid = lax.axis_index("s") * NC + lax.axis_index("c")
        base = wid * bpw
        pltpu.sync_copy(ci.at[pl.ds(base, bpw)], iv)
        copies = []
        for d in range(EMB_DIM):
            copies.append(
                pltpu.async_copy(tables[d].at[iv], dests[d], sems[d % NSEM]))
        for d in range(EMB_DIM):
            copies[d].wait()
            pltpu.sync_copy(dests[d], outs[d].at[pl.ds(base, bpw)])

    return k(*e_list, cid)


def _mlp_body(f_ref, s_ref, u_ref, *rest):
    rc_refs = rest[0:EMB_DIM]
    w1_ref, b1_ref, w2_ref, b2_ref, o_ref = rest[EMB_DIM:]
    xcT = jnp.concatenate([rc_refs[d][...][None, :] for d in range(EMB_DIM)],
                          axis=0)
    xc = jnp.transpose(xcT, (1, 0))
    h = jnp.dot(f_ref[...], w1_ref[0:IN_FEAT, :],
                preferred_element_type=jnp.float32)
    h += jnp.dot(xc, w1_ref[IN_FEAT:IN_FEAT + EMB_DIM, :],
                 preferred_element_type=jnp.float32)
    h += jnp.dot(s_ref[...], w1_ref[IN_FEAT + EMB_DIM:IN_FEAT + 2 * EMB_DIM, :],
                 preferred_element_type=jnp.float32)
    h += jnp.dot(u_ref[...], w1_ref[IN_FEAT + 2 * EMB_DIM:, :],
                 preferred_element_type=jnp.float32)
    h = jnp.maximum(h + b1_ref[...], 0.0)
    o_ref[...] = jnp.dot(h, w2_ref[...],
                         preferred_element_type=jnp.float32) + b2_ref[...]


def _mlp(features, rc_list, rs, ru, W1, b1, W2, b2):
    BLK = 4096
    grid = (BATCH // BLK,)
    d_in = IN_FEAT + 3 * EMB_DIM
    out = pl.pallas_call(
        _mlp_body,
        grid=grid,
        in_specs=(
            [
                pl.BlockSpec((BLK, IN_FEAT), lambda i: (i, 0)),
                pl.BlockSpec((BLK, EMB_DIM), lambda i: (i, 0)),
                pl.BlockSpec((BLK, EMB_DIM), lambda i: (i, 0)),
            ]
            + [pl.BlockSpec((BLK,), lambda i: (i,))] * EMB_DIM
            + [
                pl.BlockSpec((d_in, HIDDEN), lambda i: (0, 0)),
                pl.BlockSpec((1, HIDDEN), lambda i: (0, 0)),
                pl.BlockSpec((HIDDEN, 1), lambda i: (0, 0)),
                pl.BlockSpec((1, 1), lambda i: (0, 0)),
            ]
        ),
        out_specs=pl.BlockSpec((BLK, 1), lambda i: (i, 0)),
        out_shape=jax.ShapeDtypeStruct((BATCH, 1), jnp.float32),
    )(features, rs, ru, *rc_list, W1, b1.reshape(1, HIDDEN), W2,
      b2.reshape(1, 1))
    return out[:, 0]


def kernel(features, client_id, segment_id, currency_pair_id,
           client_emb, segment_emb, currency_emb, W1, b1, W2, b2):
    cid = client_id.astype(jnp.int32)
    sid = segment_id.astype(jnp.int32)
    uid = currency_pair_id.astype(jnp.int32)
    rs, ru = _sc_small(segment_emb, currency_emb, sid, uid)
    e_list = _tc_split(client_emb.T)
    rc_list = _sc_client_elem(e_list, cid)
    return _mlp(features, rc_list, rs, ru, W1, b1, W2, b2)


# speedup vs baseline: 6.2515x; 1.0109x over previous
import functools

import jax
import jax.numpy as jnp
from jax import lax
from jax.experimental import pallas as pl
from jax.experimental.pallas import tpu as pltpu
from jax.experimental.pallas import tpu_sc as plsc

BATCH = 16384
EMB_DIM = 16
IN_FEAT = 6
HIDDEN = 64
NUM_CL = 1000000


def _tc_split(t_T):
    LBLK = 131072
    grid = (pl.cdiv(NUM_CL, LBLK),)

    def body(x_ref, *o_refs):
        x = x_ref[...]
        for d in range(EMB_DIM):
            o_refs[d][...] = x[d, :]

    return pl.pallas_call(
        body,
        grid=grid,
        in_specs=[pl.BlockSpec((EMB_DIM, LBLK), lambda i: (0, i))],
        out_specs=[pl.BlockSpec((LBLK,), lambda i: (i,))] * EMB_DIM,
        out_shape=[jax.ShapeDtypeStruct((NUM_CL,), jnp.float32)] * EMB_DIM,
    )(t_T)


def _sc_small(segment_emb, currency_emb, sid, uid):
    info = plsc.get_sparse_core_info()
    NC, NS = info.num_cores, info.num_subcores
    NW = NC * NS
    bpw = BATCH // NW

    mesh = plsc.VectorSubcoreMesh(core_axis_name="c", subcore_axis_name="s")

    @functools.partial(
        pl.kernel,
        mesh=mesh,
        out_type=[jax.ShapeDtypeStruct((BATCH, EMB_DIM), jnp.float32)] * 2,
        scratch_types=[
            pltpu.VMEM((bpw,), jnp.int32),
            pltpu.VMEM((bpw,), jnp.int32),
            pltpu.VMEM((bpw, EMB_DIM), jnp.float32),
            pltpu.VMEM((bpw, EMB_DIM), jnp.float32),
            pltpu.SemaphoreType.DMA,
            pltpu.SemaphoreType.DMA,
        ],
        compiler_params=pltpu.CompilerParams(use_tc_tiling_on_sc=False),
    )
    def k(se, ue, si, ui, osg, ocu, iv1, iv2, rv1, rv2, s1, s2):
        wid = lax.axis_index("s") * NC + lax.axis_index("c")
        base = wid * bpw
        pltpu.sync_copy(si.at[pl.ds(base, bpw)], iv1)
        pltpu.sync_copy(ui.at[pl.ds(base, bpw)], iv2)
        c1 = pltpu.async_copy(se.at[iv1], rv1, s1)
        c2 = pltpu.async_copy(ue.at[iv2], rv2, s2)
        c1.wait()
        c2.wait()
        pltpu.sync_copy(rv1, osg.at[pl.ds(base, bpw)])
        pltpu.sync_copy(rv2, ocu.at[pl.ds(base, bpw)])

    return k(segment_emb, currency_emb, sid, uid)


def _sc_client_elem(e_list, cid):
    info = plsc.get_sparse_core_info()
    NC, NS = info.num_cores, info.num_subcores
    NW = NC * NS
    bpw = BATCH // NW
    NSEM = 8

    mesh = plsc.VectorSubcoreMesh(core_axis_name="c", subcore_axis_name="s")

    @functools.partial(
        pl.kernel,
        mesh=mesh,
        out_type=[jax.ShapeDtypeStruct((BATCH,), jnp.float32)] * EMB_DIM,
        scratch_types=(
            [pltpu.VMEM((bpw,), jnp.int32)]
            + [pltpu.VMEM((bpw,), jnp.float32)] * EMB_DIM
            + [pltpu.SemaphoreType.DMA] * NSEM
        ),
        compiler_params=pltpu.CompilerParams(use_tc_tiling_on_sc=False),
    )
    def k(*refs):
        tables = refs[0:EMB_DIM]
        ci = refs[EMB_DIM]
        outs = refs[EMB_DIM + 1:2 * EMB_DIM + 1]
        iv = refs[2 * EMB_DIM + 1]
        dests = refs[2 * EMB_DIM + 2:3 * EMB_DIM + 2]
        sems = refs[3 * EMB_DIM + 2:]
        wid = lax.axis_index("s") * NC + lax.axis_index("c")
        base = wid * bpw
        pltpu.sync_copy(ci.at[pl.ds(base, bpw)], iv)
        copies = []
        for d in range(EMB_DIM):
            copies.append(
                pltpu.async_copy(tables[d].at[iv], dests[d], sems[d % NSEM]))
        for d in range(EMB_DIM):
            copies[d].wait()
            pltpu.sync_copy(dests[d], outs[d].at[pl.ds(base, bpw)])

    return k(*e_list, cid)


def _mlp_body(f_ref, s_ref, u_ref, *rest):
    rc_refs = rest[0:EMB_DIM]
    w1_ref, b1_ref, w2_ref, b2_ref, o_ref = rest[EMB_DIM:]
    xcT = jnp.concatenate([rc_refs[d][...][None, :] for d in range(EMB_DIM)],
                          axis=0)
    xc = jnp.transpose(xcT, (1, 0))
    h = jnp.dot(f_ref[...], w1_ref[0:IN_FEAT, :],
                preferred_element_type=jnp.float32)
    h += jnp.dot(xc, w1_ref[IN_FEAT:IN_FEAT + EMB_DIM, :],
                 preferred_element_type=jnp.float32)
    h += jnp.dot(s_ref[...], w1_ref[IN_FEAT + EMB_DIM:IN_FEAT + 2 * EMB_DIM, :],
                 preferred_element_type=jnp.float32)
    h += jnp.dot(u_ref[...], w1_ref[IN_FEAT + 2 * EMB_DIM:, :],
                 preferred_element_type=jnp.float32)
    h = jnp.maximum(h + b1_ref[...], 0.0)
    o_ref[...] = jnp.dot(h, w2_ref[...],
                         preferred_element_type=jnp.float32) + b2_ref[...]


def _mlp(features, rc_list, rs, ru, W1, b1, W2, b2):
    BLK = 8192
    grid = (BATCH // BLK,)
    d_in = IN_FEAT + 3 * EMB_DIM
    out = pl.pallas_call(
        _mlp_body,
        grid=grid,
        in_specs=(
            [
                pl.BlockSpec((BLK, IN_FEAT), lambda i: (i, 0)),
                pl.BlockSpec((BLK, EMB_DIM), lambda i: (i, 0)),
                pl.BlockSpec((BLK, EMB_DIM), lambda i: (i, 0)),
            ]
            + [pl.BlockSpec((BLK,), lambda i: (i,))] * EMB_DIM
            + [
                pl.BlockSpec((d_in, HIDDEN), lambda i: (0, 0)),
                pl.BlockSpec((1, HIDDEN), lambda i: (0, 0)),
                pl.BlockSpec((HIDDEN, 1), lambda i: (0, 0)),
                pl.BlockSpec((1, 1), lambda i: (0, 0)),
            ]
        ),
        out_specs=pl.BlockSpec((BLK, 1), lambda i: (i, 0)),
        out_shape=jax.ShapeDtypeStruct((BATCH, 1), jnp.float32),
    )(features, rs, ru, *rc_list, W1, b1.reshape(1, HIDDEN), W2,
      b2.reshape(1, 1))
    return out[:, 0]


def kernel(features, client_id, segment_id, currency_pair_id,
           client_emb, segment_emb, currency_emb, W1, b1, W2, b2):
    cid = client_id.astype(jnp.int32)
    sid = segment_id.astype(jnp.int32)
    uid = currency_pair_id.astype(jnp.int32)
    rs, ru = _sc_small(segment_emb, currency_emb, sid, uid)
    e_list = _tc_split(client_emb.T)
    rc_list = _sc_client_elem(e_list, cid)
    return _mlp(features, rc_list, rs, ru, W1, b1, W2, b2)
